# Initial kernel scaffold; baseline (speedup 1.0000x reference)
import jax
import jax.numpy as jnp
from jax.experimental import pallas as pl


def kernel(all_deletion_ids, all_insertion_ids, all_subs_ids, action_scores):
    raise NotImplementedError("write your pallas kernel here")


# speedup vs baseline: 12.2453x; 12.2453x over previous
import functools

import jax
import jax.numpy as jnp
from jax import lax
from jax.experimental import pallas as pl
from jax.experimental.pallas import tpu as pltpu
from jax.experimental.pallas import tpu_sc as plsc

_B = 128
_SRC = 20
_TGT = 20
_NC = 288
_ND = _SRC + _TGT - 1
_NEG = -1e30

_NW = 32
_CHUNKS = 78
_CHUNK = 120


def _build_indices(all_deletion_ids, all_insertion_ids, all_subs_ids):
    d = jnp.arange(_ND)[:, None]
    t = jnp.arange(_SRC)[None, :]
    vc = jnp.clip(d - t, 0, _TGT - 1)
    ins_ids_d = jnp.take(all_insertion_ids.T, vc, axis=0)
    del_ids_d = jnp.broadcast_to(all_deletion_ids.T[None], (_ND, _SRC, _B))
    sub_ids_d = jnp.take(all_subs_ids.reshape(_B, _SRC * _TGT).T,
                         t * _TGT + vc, axis=0)
    b = jnp.arange(_B)[None, None, :]
    base = ((b * _SRC + t[:, :, None]) * _TGT + vc[:, :, None]) * _NC
    idx = jnp.stack([base + ins_ids_d, base + del_ids_d, base + sub_ids_d])
    return idx.astype(jnp.int32)


def _sc_gather_body(table_hbm, idx_hbm, out_hbm, idx_v, out_v, sem):
    wid = lax.axis_index("s") * 2 + lax.axis_index("c")
    pltpu.sync_copy(idx_hbm.at[wid], idx_v)

    def chunk(i, carry):
        pltpu.async_copy(table_hbm.at[idx_v.at[i]], out_v.at[i], sem).wait()
        return carry

    lax.fori_loop(0, _CHUNKS, chunk, 0)
    pltpu.sync_copy(out_v, out_hbm.at[wid])


_sc_gather = functools.partial(
    pl.kernel,
    out_type=jax.ShapeDtypeStruct((_NW, _CHUNKS, _CHUNK), jnp.float32),
    mesh=plsc.VectorSubcoreMesh(core_axis_name="c", subcore_axis_name="s"),
    scratch_types=[
        pltpu.VMEM((_CHUNKS, _CHUNK), jnp.int32),
        pltpu.VMEM((_CHUNKS, _CHUNK), jnp.float32),
        pltpu.SemaphoreType.DMA,
    ],
)(_sc_gather_body)


def _dp_body(scores_ref, out_ref):
    prevprev = jnp.full((_SRC, _B), _NEG, jnp.float32)
    prev = jnp.zeros((_SRC, _B), jnp.float32)
    out_ref[0, 0, :] = prev[0, :]
    for d in range(1, _ND):
        lo = max(0, d - (_TGT - 1))
        hi = min(d, _SRC - 1)
        tt = lax.broadcasted_iota(jnp.int32, (_SRC, _B), 0)
        m_ins = (tt >= lo) & (tt <= min(d - 1, _SRC - 1))
        m_del = (tt >= max(1, lo)) & (tt <= hi)
        m_sub = (tt >= max(1, lo)) & (tt <= min(d - 1, _SRC - 1))
        ins = scores_ref[0, d]
        dl = scores_ref[1, d]
        sb = scores_ref[2, d]
        neg_row = jnp.full((1, _B), _NEG, jnp.float32)
        prev_sh = jnp.concatenate([neg_row, prev[:-1]], axis=0)
        pp_sh = jnp.concatenate([neg_row, prevprev[:-1]], axis=0)
        t_ins = jnp.where(m_ins, ins + prev, _NEG)
        t_del = jnp.where(m_del, dl + prev_sh, _NEG)
        t_sub = jnp.where(m_sub, sb + pp_sh, _NEG)
        m = jnp.maximum(jnp.maximum(t_ins, t_del), t_sub)
        a = m + jnp.log(jnp.exp(t_ins - m) + jnp.exp(t_del - m) + jnp.exp(t_sub - m))
        for t in range(lo, hi + 1):
            out_ref[t, d - t, :] = a[t, :]
        prevprev, prev = prev, a


def kernel(all_deletion_ids, all_insertion_ids, all_subs_ids, action_scores):
    idx = _build_indices(all_deletion_ids, all_insertion_ids, all_subs_ids)
    table = action_scores.reshape(-1)
    gathered = _sc_gather(table, idx.reshape(_NW, _CHUNKS, _CHUNK))
    gathered = gathered.reshape(3, _ND, _SRC, _B)
    out = pl.pallas_call(
        _dp_body,
        out_shape=jax.ShapeDtypeStruct((_SRC, _TGT, _B), jnp.float32),
    )(gathered)
    return out.transpose(2, 0, 1)


# speedup vs baseline: 14.7491x; 1.2045x over previous
import functools

import jax
import jax.numpy as jnp
from jax import lax
from jax.experimental import pallas as pl
from jax.experimental.pallas import tpu as pltpu
from jax.experimental.pallas import tpu_sc as plsc

_B = 128
_SRC = 20
_TGT = 20
_NC = 288
_ND = _SRC + _TGT - 1
_NEG = -1e30

_NW = 32
_CHUNKS = 78
_CHUNK = 120
_GROUP = 13


def _build_indices(all_deletion_ids, all_insertion_ids, all_subs_ids):
    d = jnp.arange(_ND)[:, None]
    t = jnp.arange(_SRC)[None, :]
    vc = jnp.clip(d - t, 0, _TGT - 1)
    ins_t = all_insertion_ids.T.astype(jnp.int32)
    pad = jnp.zeros((_SRC - 1, _B), jnp.int32)
    ins_p = jnp.concatenate([pad, ins_t, pad], axis=0)
    ins_ids_d = jnp.stack(
        [lax.slice_in_dim(ins_p, _SRC - 1 - tt, _SRC - 1 - tt + _ND)
         for tt in range(_SRC)], axis=1)
    del_ids_d = jnp.broadcast_to(
        all_deletion_ids.T.astype(jnp.int32)[None], (_ND, _SRC, _B))
    sub_t = all_subs_ids.reshape(_B, _SRC * _TGT).T.astype(jnp.int32)
    sub_ids_d = jnp.stack(
        [lax.slice_in_dim(sub_t, tt * (_TGT - 1), tt * (_TGT - 1) + _ND)
         for tt in range(_SRC)], axis=1)
    b = jnp.arange(_B)[None, None, :]
    base = (((b * _SRC + t[:, :, None]) * _TGT + vc[:, :, None]) * _NC).astype(jnp.int32)
    idx = jnp.stack([base + ins_ids_d, base + del_ids_d, base + sub_ids_d])
    return idx


def _sc_gather_body(table_hbm, idx_hbm, out_hbm, idx_v, out_v, sem):
    wid = lax.axis_index("s") * 2 + lax.axis_index("c")
    pltpu.sync_copy(idx_hbm.at[wid], idx_v)

    def group(g, carry):
        descs = []
        for j in range(_GROUP):
            i = g * _GROUP + j
            descs.append(
                pltpu.async_copy(table_hbm.at[idx_v.at[i]], out_v.at[i], sem))
        for dsc in descs:
            dsc.wait()
        return carry

    lax.fori_loop(0, _CHUNKS // _GROUP, group, 0)
    pltpu.sync_copy(out_v, out_hbm.at[wid])


_sc_gather = functools.partial(
    pl.kernel,
    out_type=jax.ShapeDtypeStruct((_NW, _CHUNKS, _CHUNK), jnp.float32),
    mesh=plsc.VectorSubcoreMesh(core_axis_name="c", subcore_axis_name="s"),
    scratch_types=[
        pltpu.VMEM((_CHUNKS, _CHUNK), jnp.int32),
        pltpu.VMEM((_CHUNKS, _CHUNK), jnp.float32),
        pltpu.SemaphoreType.DMA,
    ],
)(_sc_gather_body)


def _dp_body(scores_ref, out_ref):
    prevprev = jnp.full((_SRC, _B), _NEG, jnp.float32)
    prev = jnp.zeros((_SRC, _B), jnp.float32)
    out_ref[0, 0, :] = prev[0, :]
    for d in range(1, _ND):
        lo = max(0, d - (_TGT - 1))
        hi = min(d, _SRC - 1)
        tt = lax.broadcasted_iota(jnp.int32, (_SRC, _B), 0)
        m_ins = (tt >= lo) & (tt <= min(d - 1, _SRC - 1))
        m_del = (tt >= max(1, lo)) & (tt <= hi)
        m_sub = (tt >= max(1, lo)) & (tt <= min(d - 1, _SRC - 1))
        ins = scores_ref[0, d]
        dl = scores_ref[1, d]
        sb = scores_ref[2, d]
        neg_row = jnp.full((1, _B), _NEG, jnp.float32)
        prev_sh = jnp.concatenate([neg_row, prev[:-1]], axis=0)
        pp_sh = jnp.concatenate([neg_row, prevprev[:-1]], axis=0)
        t_ins = jnp.where(m_ins, ins + prev, _NEG)
        t_del = jnp.where(m_del, dl + prev_sh, _NEG)
        t_sub = jnp.where(m_sub, sb + pp_sh, _NEG)
        m = jnp.maximum(jnp.maximum(t_ins, t_del), t_sub)
        a = m + jnp.log(jnp.exp(t_ins - m) + jnp.exp(t_del - m) + jnp.exp(t_sub - m))
        for t in range(lo, hi + 1):
            out_ref[t, d - t, :] = a[t, :]
        prevprev, prev = prev, a


def kernel(all_deletion_ids, all_insertion_ids, all_subs_ids, action_scores):
    idx = _build_indices(all_deletion_ids, all_insertion_ids, all_subs_ids)
    table = action_scores.reshape(-1)
    gathered = _sc_gather(table, idx.reshape(_NW, _CHUNKS, _CHUNK))
    gathered = gathered.reshape(3, _ND, _SRC, _B)
    out = pl.pallas_call(
        _dp_body,
        out_shape=jax.ShapeDtypeStruct((_SRC, _TGT, _B), jnp.float32),
    )(gathered)
    return out.transpose(2, 0, 1)


# speedup vs baseline: 14.7701x; 1.0014x over previous
import functools

import jax
import jax.numpy as jnp
from jax import lax
from jax.experimental import pallas as pl
from jax.experimental.pallas import tpu as pltpu
from jax.experimental.pallas import tpu_sc as plsc

_B = 128
_SRC = 20
_TGT = 20
_NC = 288
_ND = _SRC + _TGT - 1
_NEG = -1e30

_NW = 32
_CHUNKS = 78
_CHUNK = 120
_GROUP = 13


def _build_indices(all_deletion_ids, all_insertion_ids, all_subs_ids):
    d = jnp.arange(_ND)[:, None]
    t = jnp.arange(_SRC)[None, :]
    vc = jnp.clip(d - t, 0, _TGT - 1)
    ins_t = all_insertion_ids.T.astype(jnp.int32)
    pad = jnp.zeros((_SRC - 1, _B), jnp.int32)
    ins_p = jnp.concatenate([pad, ins_t, pad], axis=0)
    ins_ids_d = jnp.stack(
        [lax.slice_in_dim(ins_p, _SRC - 1 - tt, _SRC - 1 - tt + _ND)
         for tt in range(_SRC)], axis=1)
    del_ids_d = jnp.broadcast_to(
        all_deletion_ids.T.astype(jnp.int32)[None], (_ND, _SRC, _B))
    sub_t = all_subs_ids.reshape(_B, _SRC * _TGT).T.astype(jnp.int32)
    sub_ids_d = jnp.stack(
        [lax.slice_in_dim(sub_t, tt * (_TGT - 1), tt * (_TGT - 1) + _ND)
         for tt in range(_SRC)], axis=1)
    b = jnp.arange(_B)[None, None, :]
    base = (((b * _SRC + t[:, :, None]) * _TGT + vc[:, :, None]) * _NC).astype(jnp.int32)
    idx = jnp.stack([base + ins_ids_d, base + del_ids_d, base + sub_ids_d])
    return idx


def _sc_gather_body(table_hbm, idx_hbm, out_hbm, idx_v, out_v, sem):
    wid = lax.axis_index("s") * 2 + lax.axis_index("c")
    pltpu.sync_copy(idx_hbm.at[wid], idx_v)

    def group(g, carry):
        descs = []
        for j in range(_GROUP):
            i = g * _GROUP + j
            descs.append(
                pltpu.async_copy(table_hbm.at[idx_v.at[i]], out_v.at[i], sem))
        for dsc in descs:
            dsc.wait()
        return carry

    lax.fori_loop(0, _CHUNKS // _GROUP, group, 0)
    pltpu.sync_copy(out_v, out_hbm.at[wid])


_sc_gather = functools.partial(
    pl.kernel,
    out_type=jax.ShapeDtypeStruct((_NW, _CHUNKS, _CHUNK), jnp.float32),
    mesh=plsc.VectorSubcoreMesh(core_axis_name="c", subcore_axis_name="s"),
    scratch_types=[
        pltpu.VMEM((_CHUNKS, _CHUNK), jnp.int32),
        pltpu.VMEM((_CHUNKS, _CHUNK), jnp.float32),
        pltpu.SemaphoreType.DMA,
    ],
)(_sc_gather_body)


def _dp_body(scores_ref, out_ref):
    prevprev = jnp.full((_SRC, _B), _NEG, jnp.float32)
    prev = jnp.zeros((_SRC, _B), jnp.float32)
    out_ref[0, 0, :] = prev[0, :]
    for d in range(1, _ND):
        lo = max(0, d - (_TGT - 1))
        hi = min(d, _SRC - 1)
        tt = lax.broadcasted_iota(jnp.int32, (_SRC, _B), 0)
        m_ins = (tt >= lo) & (tt <= min(d - 1, _SRC - 1))
        m_del = (tt >= max(1, lo)) & (tt <= hi)
        m_sub = (tt >= max(1, lo)) & (tt <= min(d - 1, _SRC - 1))
        ins = scores_ref[0, d]
        dl = scores_ref[1, d]
        sb = scores_ref[2, d]
        neg_row = jnp.full((1, _B), _NEG, jnp.float32)
        prev_sh = jnp.concatenate([neg_row, prev[:-1]], axis=0)
        pp_sh = jnp.concatenate([neg_row, prevprev[:-1]], axis=0)
        t_ins = jnp.where(m_ins, ins + prev, _NEG)
        t_del = jnp.where(m_del, dl + prev_sh, _NEG)
        t_sub = jnp.where(m_sub, sb + pp_sh, _NEG)
        m = jnp.maximum(jnp.maximum(t_ins, t_del), t_sub)
        a = m + jnp.log(jnp.exp(t_ins - m) + jnp.exp(t_del - m) + jnp.exp(t_sub - m))
        for t in range(lo, hi + 1):
            out_ref[t, d - t, :] = a[t, :]
        prevprev, prev = prev, a


def kernel(all_deletion_ids, all_insertion_ids, all_subs_ids, action_scores):
    idx = _build_indices(all_deletion_ids, all_insertion_ids, all_subs_ids)
    table = lax.optimization_barrier(
        action_scores.reshape(_B * _SRC * _TGT * _NC // 128, 128)).reshape(-1)
    gathered = _sc_gather(table, idx.reshape(_NW, _CHUNKS, _CHUNK))
    gathered = gathered.reshape(3, _ND, _SRC, _B)
    out = pl.pallas_call(
        _dp_body,
        out_shape=jax.ShapeDtypeStruct((_SRC, _TGT, _B), jnp.float32),
    )(gathered)
    return out.transpose(2, 0, 1)
